# baseline (speedup 1.0000x reference)
import jax
import jax.numpy as jnp
from jax.experimental import pallas as pl
from jax.experimental.pallas import tpu as pltpu


def _cummax_body(x_ref, o_ref):
    nb, _, h, w = x_ref.shape
    rows = jax.lax.broadcasted_iota(jnp.int32, (8, 1), 0)
    neg_inf = jnp.float32(-jnp.inf)

    def tile_step(v, carrys):
        new_carrys = []
        for ib in range(nb):
            t = x_ref[ib, 0, pl.ds(v * 8, 8), :]
            t3 = t
            for s in (1, 2, 4):
                r = pltpu.roll(t3, s, axis=0)
                t3 = jnp.maximum(t3, jnp.where(rows >= s, r, neg_inf))
            tot = t
            for s in (1, 2, 4):
                tot = jnp.maximum(tot, pltpu.roll(tot, s, axis=0))
            o_ref[ib, 0, pl.ds(v * 8, 8), :] = jnp.maximum(t3, carrys[ib])
            new_carrys.append(jnp.maximum(carrys[ib], tot))
        return tuple(new_carrys)

    carry0 = jnp.full((8, w), neg_inf, jnp.float32)
    jax.lax.fori_loop(0, h // 8, tile_step, (carry0,) * nb, unroll=4)


def kernel(x):
    b, c, h, w = x.shape
    nb = 2 if b % 2 == 0 else 1
    return pl.pallas_call(
        _cummax_body,
        grid=(b // nb,),
        in_specs=[pl.BlockSpec((nb, c, h, w), lambda i: (i, 0, 0, 0))],
        out_specs=pl.BlockSpec((nb, c, h, w), lambda i: (i, 0, 0, 0)),
        out_shape=jax.ShapeDtypeStruct(x.shape, x.dtype),
        compiler_params=pltpu.CompilerParams(
            dimension_semantics=("parallel",),
        ),
    )(x)


# speedup vs baseline: 1.0859x; 1.0859x over previous
import jax
import jax.numpy as jnp
from jax.experimental import pallas as pl
from jax.experimental.pallas import tpu as pltpu


def _cummax_body(x_ref, o_ref):
    nb, _, h, w = x_ref.shape
    rows = jax.lax.broadcasted_iota(jnp.int32, (8, 1), 0)
    neg_inf = jnp.float32(-jnp.inf)
    for ib in range(nb):
        def tile_step(v, carry):
            t = x_ref[ib, 0, pl.ds(v * 8, 8), :]
            t3 = t
            for s in (1, 2, 4):
                r = pltpu.roll(t3, s, axis=0)
                t3 = jnp.maximum(t3, jnp.where(rows >= s, r, neg_inf))
            out = jnp.maximum(t3, carry)
            o_ref[ib, 0, pl.ds(v * 8, 8), :] = out
            return jnp.broadcast_to(out[7:8], (8, w))

        carry0 = jnp.full((8, w), neg_inf, jnp.float32)
        jax.lax.fori_loop(0, h // 8, tile_step, carry0, unroll=4)


def kernel(x):
    b, c, h, w = x.shape
    nb = 2 if b % 2 == 0 else 1
    return pl.pallas_call(
        _cummax_body,
        grid=(b // nb,),
        in_specs=[pl.BlockSpec((nb, c, h, w), lambda i: (i, 0, 0, 0))],
        out_specs=pl.BlockSpec((nb, c, h, w), lambda i: (i, 0, 0, 0)),
        out_shape=jax.ShapeDtypeStruct(x.shape, x.dtype),
        compiler_params=pltpu.CompilerParams(
            dimension_semantics=("parallel",),
        ),
    )(x)


# speedup vs baseline: 1.0919x; 1.0056x over previous
import jax
import jax.numpy as jnp
from jax.experimental import pallas as pl
from jax.experimental.pallas import tpu as pltpu


def _cummax_body(x_ref, o_ref):
    nb, _, h, w = x_ref.shape
    rows = jax.lax.broadcasted_iota(jnp.int32, (8, 1), 0)
    neg_inf = jnp.float32(-jnp.inf)
    for ib in range(nb):
        def tile_step(v, carry):
            t = x_ref[ib, 0, pl.ds(v * 8, 8), :]
            t3 = t
            for s in (1, 2, 4):
                r = pltpu.roll(t3, s, axis=0)
                t3 = jnp.maximum(t3, jnp.where(rows >= s, r, neg_inf))
            out = jnp.maximum(t3, carry)
            o_ref[ib, 0, pl.ds(v * 8, 8), :] = out
            return jnp.broadcast_to(out[7:8], (8, w))

        carry0 = jnp.full((8, w), neg_inf, jnp.float32)
        jax.lax.fori_loop(0, h // 8, tile_step, carry0, unroll=8)


def kernel(x):
    b, c, h, w = x.shape
    nb = 2 if b % 2 == 0 else 1
    return pl.pallas_call(
        _cummax_body,
        grid=(b // nb,),
        in_specs=[pl.BlockSpec((nb, c, h, w), lambda i: (i, 0, 0, 0))],
        out_specs=pl.BlockSpec((nb, c, h, w), lambda i: (i, 0, 0, 0)),
        out_shape=jax.ShapeDtypeStruct(x.shape, x.dtype),
        compiler_params=pltpu.CompilerParams(
            dimension_semantics=("parallel",),
        ),
    )(x)


# speedup vs baseline: 1.0972x; 1.0048x over previous
import jax
import jax.numpy as jnp
from jax.experimental import pallas as pl
from jax.experimental.pallas import tpu as pltpu


def _cummax_body(x_ref, o_ref):
    nb, _, h, w = x_ref.shape
    rows = jax.lax.broadcasted_iota(jnp.int32, (8, 1), 0)
    neg_inf = jnp.float32(-jnp.inf)
    for ib in range(nb):
        def tile_step(v, carry):
            t = x_ref[ib, 0, pl.ds(v * 8, 8), :]
            t3 = t
            for s in (1, 2, 4):
                r = pltpu.roll(t3, s, axis=0)
                t3 = jnp.maximum(t3, jnp.where(rows >= s, r, neg_inf))
            out = jnp.maximum(t3, carry)
            o_ref[ib, 0, pl.ds(v * 8, 8), :] = out
            return jnp.broadcast_to(out[7:8], (8, w))

        carry0 = jnp.full((8, w), neg_inf, jnp.float32)
        jax.lax.fori_loop(0, h // 8, tile_step, carry0, unroll=16)


def kernel(x):
    b, c, h, w = x.shape
    nb = 2 if b % 2 == 0 else 1
    return pl.pallas_call(
        _cummax_body,
        grid=(b // nb,),
        in_specs=[pl.BlockSpec((nb, c, h, w), lambda i: (i, 0, 0, 0))],
        out_specs=pl.BlockSpec((nb, c, h, w), lambda i: (i, 0, 0, 0)),
        out_shape=jax.ShapeDtypeStruct(x.shape, x.dtype),
        compiler_params=pltpu.CompilerParams(
            dimension_semantics=("parallel",),
        ),
    )(x)
